# baseline (speedup 1.0000x reference)
import functools

import jax
import jax.numpy as jnp
from jax import lax
from jax.experimental import pallas as pl
from jax.experimental.pallas import tpu as pltpu
from jax.experimental.pallas import tpu_sc as plsc

NC = 2
NS = 16
NW = NC * NS
CHUNK = 128

def _gather_body(nchunk, uids, iids, utab, itab, uout, iout,
                 uidx, iidx, urows, irows, sem):
    wid = lax.axis_index("s") * NC + lax.axis_index("c")
    pltpu.sync_copy(uids.at[pl.ds(wid * nchunk, nchunk)], uidx)
    pltpu.sync_copy(iids.at[pl.ds(wid * nchunk, nchunk)], iidx)
    copies = []
    for j in range(nchunk):
        copies.append(pltpu.async_copy(utab.at[uidx.at[j]], urows.at[j], sem))
        copies.append(pltpu.async_copy(itab.at[iidx.at[j]], irows.at[j], sem))
    for c in copies:
        c.wait()
    pltpu.sync_copy(urows, uout.at[pl.ds(wid * nchunk, nchunk)])
    pltpu.sync_copy(irows, iout.at[pl.ds(wid * nchunk, nchunk)])


def _sc_gather(user_ids, item_ids, user_table, item_table):
    B = user_ids.shape[0]
    D = user_table.shape[1]
    bpw = B // NW
    nchunk = bpw // CHUNK
    uids2 = user_ids.reshape(B // CHUNK, CHUNK).astype(jnp.int32)
    iids2 = item_ids.reshape(B // CHUNK, CHUNK).astype(jnp.int32)
    body = functools.partial(_gather_body, nchunk)
    out3 = jax.ShapeDtypeStruct((B // CHUNK, CHUNK, D), jnp.float32)
    mesh = plsc.VectorSubcoreMesh(
        core_axis_name="c", subcore_axis_name="s", num_cores=NC, num_subcores=NS
    )
    k = pl.kernel(
        body,
        out_type=(out3, out3),
        mesh=mesh,
        compiler_params=pltpu.CompilerParams(use_tc_tiling_on_sc=False),
        scratch_types=[
            pltpu.VMEM((nchunk, CHUNK), jnp.int32),
            pltpu.VMEM((nchunk, CHUNK), jnp.int32),
            pltpu.VMEM((nchunk, CHUNK, D), jnp.float32),
            pltpu.VMEM((nchunk, CHUNK, D), jnp.float32),
            pltpu.SemaphoreType.DMA,
        ],
    )
    u3, i3 = k(uids2, iids2, user_table, item_table)
    return u3.reshape(B, D), i3.reshape(B, D)


def _mlp_body(u_ref, i_ref, w1u_ref, w1i_ref, b1_ref, w2_ref, b2_ref,
              w3_ref, b3_ref, o_ref):
    h = (
        jnp.dot(u_ref[...], w1u_ref[...], preferred_element_type=jnp.float32)
        + jnp.dot(i_ref[...], w1i_ref[...], preferred_element_type=jnp.float32)
        + b1_ref[...]
    )
    h = jnp.maximum(h, 0.0)
    h = jnp.dot(h, w2_ref[...], preferred_element_type=jnp.float32) + b2_ref[...]
    h = jnp.maximum(h, 0.0)
    z = jnp.sum(h * w3_ref[...], axis=1) + b3_ref[...]
    o_ref[...] = 1.0 / (1.0 + jnp.exp(-z))


def _tc_mlp(u, i, W1, b1, W2, b2, W3, b3):
    B, D = u.shape
    BT = 2048
    w1u = W1[:, :D].T
    w1i = W1[:, D:].T
    w2 = W2.T
    w3 = W3[0]
    grid = (B // BT,)
    return pl.pallas_call(
        _mlp_body,
        grid=grid,
        in_specs=[
            pl.BlockSpec((BT, D), lambda g: (g, 0)),
            pl.BlockSpec((BT, D), lambda g: (g, 0)),
            pl.BlockSpec(w1u.shape, lambda g: (0, 0)),
            pl.BlockSpec(w1i.shape, lambda g: (0, 0)),
            pl.BlockSpec(b1.shape, lambda g: (0,)),
            pl.BlockSpec(w2.shape, lambda g: (0, 0)),
            pl.BlockSpec(b2.shape, lambda g: (0,)),
            pl.BlockSpec(w3.shape, lambda g: (0,)),
            pl.BlockSpec(b3.shape, lambda g: (0,)),
        ],
        out_specs=pl.BlockSpec((BT,), lambda g: (g,)),
        out_shape=jax.ShapeDtypeStruct((B,), jnp.float32),
    )(u, i, w1u, w1i, b1, w2, b2, w3, b3)


def kernel(user_ids, item_ids, user_table, item_table, W1, b1, W2, b2, W3, b3):
    u, i = _sc_gather(user_ids, item_ids, user_table, item_table)
    return _tc_mlp(u, i, W1, b1, W2, b2, W3, b3)
